# Initial kernel scaffold; baseline (speedup 1.0000x reference)
import jax
import jax.numpy as jnp
from jax.experimental import pallas as pl


def kernel(node_feats, edge_feats, edge_index, graph_ids, params):
    raise NotImplementedError("write your pallas kernel here")


# speedup vs baseline: 7.6359x; 7.6359x over previous
import functools

import jax
import jax.numpy as jnp
from jax import lax
from jax.experimental import pallas as pl
from jax.experimental.pallas import tpu as pltpu
from jax.experimental.pallas import tpu_sc as plsc

N = 10000
E = 160000
F = 128
EF = 16
G = 200
NG = 64
PH = 128

NP = 10240
GP = 256
GW = 128
NC = 2
NS = 16
NW = NC * NS
LANES = 16
CHUNK = 128
NCHUNK = E // CHUNK
STEPS = (NCHUNK + NW - 1) // NW
RPS = NP // NS

_mesh = plsc.VectorSubcoreMesh(core_axis_name="c", subcore_axis_name="s",
                               num_cores=NC, num_subcores=NS)
_sc_params = pltpu.CompilerParams(use_tc_tiling_on_sc=False,
                                  needs_layout_passes=False)
_sc_params_tiled = pltpu.CompilerParams(use_tc_tiling_on_sc=True,
                                        needs_layout_passes=False)

f32 = jnp.float32
i32 = jnp.int32


def _dot(a, b):
    return lax.dot_general(a, b, (((1,), (0,)), ((), ())),
                           preferred_element_type=f32)


def _seg_dot(onehot, x):
    return lax.dot_general(onehot, x, (((0,), (0,)), ((), ())),
                           precision=lax.Precision.HIGHEST,
                           preferred_element_type=f32)


def _dot_hi(a, b):
    return lax.dot_general(a, b, (((1,), (0,)), ((), ())),
                           precision=lax.Precision.HIGHEST,
                           preferred_element_type=f32)


def _bcast_dot(onehot, g):
    return lax.dot_general(onehot, g, (((1,), (0,)), ((), ())),
                           precision=lax.Precision.HIGHEST,
                           preferred_element_type=f32)


def _lrelu(x):
    return jnp.where(x >= 0, x, 0.01 * x)


def _elu(x):
    return jnp.where(x > 0, x, jnp.exp(jnp.minimum(x, 0.0)) - 1.0)


def _sigmoid(x):
    return 1.0 / (1.0 + jnp.exp(-x))


def _gru_block(x, h, Wi, Wh, bi, bh):
    gi = _dot(x, Wi) + bi
    gh = _dot(h, Wh) + bh
    r = _sigmoid(gi[:, 0:G] + gh[:, 0:G])
    z = _sigmoid(gi[:, G:2 * G] + gh[:, G:2 * G])
    n = jnp.tanh(gi[:, 2 * G:] + r * gh[:, 2 * G:])
    return (1.0 - z) * n + z * h


def _full(shape):
    return pl.BlockSpec(shape, lambda *_: tuple(0 for _ in shape))


def _rows(bn, cols):
    return pl.BlockSpec((bn, cols), lambda i: (i, 0))


BN = 512
BE = 1000


def _tc1_body(nf, Wpn, bpn, A, wtop, bpe2, hv, np1, sn):
    hv_ = _lrelu(_dot(nf[...], Wpn[...]) + bpn[...])
    hv[...] = hv_
    np1[...] = _dot(nf[...], A[...])
    sn[...] = _dot(hv_, wtop[...]) + bpe2[...]


def _tc1(nf_p, Wpn, bpn, A_pad, wtop, bpe2):
    return pl.pallas_call(
        _tc1_body,
        grid=(NP // BN,),
        in_specs=[_rows(BN, F), _full((F, G)), _full((1, G)), _full((F, GP)),
                  _full((G, 1)), _full((1, 1))],
        out_specs=[_rows(BN, G), _rows(BN, GP), _rows(BN, 1)],
        out_shape=[jax.ShapeDtypeStruct((NP, G), f32),
                   jax.ShapeDtypeStruct((NP, GP), f32),
                   jax.ShapeDtypeStruct((NP, 1), f32)],
        name="tc1_node_prep",
    )(nf_p, Wpn, bpn, A_pad, wtop, bpe2)


def _tc3_body(gnp1, ef, snd, Bm, bpe1, wbot, Wet, bet, w):
    he1 = _lrelu(gnp1[...] + _dot(ef[...], Bm[...]) + bpe1[...])
    se = _dot(he1, wbot[...])
    e = jnp.exp(_lrelu(snd[...] + se))
    m = _dot(he1, Wet[...]) + bet[...]
    w[...] = e * m
    w[:, G:G + 1] = e


def _tc3(gnp1, ef, snd, Bm_pad, bpe1_pad, wbot_pad, Wet_pad, bet_pad):
    return pl.pallas_call(
        _tc3_body,
        grid=(E // BE,),
        in_specs=[_rows(BE, GP), _rows(BE, EF), _rows(BE, 1),
                  _full((EF, GP)), _full((1, GP)), _full((GP, 1)),
                  _full((GP, GP)), _full((1, GP))],
        out_specs=_rows(BE, GP),
        out_shape=jax.ShapeDtypeStruct((E, GP), f32),
        name="tc3_edge_ctx",
    )(gnp1, ef, snd, Bm_pad, bpe1_pad, wbot_pad, Wet_pad, bet_pad)


def _tc5_body(wsum, hv, Wi, Wh, bi, bh, vd, vs, bl1, Wpn2, bpn2,
              h, hp2, sd2, ss2):
    s = wsum[:, G:G + 1]
    inv = 1.0 / (s + 1e-9)
    c = wsum[:, :G] * inv
    h_ = jnp.maximum(_gru_block(_elu(c), hv[...], Wi[...], Wh[...],
                                bi[...], bh[...]), 0.0)
    h[:, :G] = h_
    h[:, G:] = jnp.zeros((BN, GP - G), f32)
    hp2[:, :G] = _dot(h_, Wpn2[...]) + bpn2[...]
    hp2[:, G:] = jnp.zeros((BN, GP - G), f32)
    hp2[:, G:G + 1] = jnp.ones((BN, 1), f32)
    sd2[...] = _dot(h_, vd[...]) + bl1[...]
    ss2[...] = _dot(h_, vs[...])


def _tc5(wsum, hv, Wi, Wh, bi, bh, vd, vs, bl1, Wpn2, bpn2):
    return pl.pallas_call(
        _tc5_body,
        grid=(NP // BN,),
        in_specs=[_rows(BN, GP),
                  _rows(BN, G),
                  _full((G, 3 * G)), _full((G, 3 * G)), _full((1, 3 * G)),
                  _full((1, 3 * G)), _full((G, 1)), _full((G, 1)),
                  _full((1, 1)), _full((G, G)), _full((1, G))],
        out_specs=[_rows(BN, GP), _rows(BN, GP), _rows(BN, 1), _rows(BN, 1)],
        out_shape=[jax.ShapeDtypeStruct((NP, GP), f32),
                   jax.ShapeDtypeStruct((NP, GP), f32),
                   jax.ShapeDtypeStruct((NP, 1), f32),
                   jax.ShapeDtypeStruct((NP, 1), f32)],
        name="tc5_layer0_update",
    )(wsum, hv, Wi, Wh, bi, bh, vd, vs, bl1, Wpn2, bpn2)


def _tc7a_body(wsum, h, gid, Wi, Wh, bi, bh,
               un0, un1, h2, zn0, zn1, g0):
    s = wsum[:, G:G + 1]
    inv = 1.0 / (s + 1e-9)
    c = wsum[:, :G] * inv
    h2_ = jnp.maximum(_gru_block(_elu(c), h[:, :G], Wi[...], Wh[...],
                                 bi[...], bh[...]), 0.0)
    h2[...] = h2_
    zn0[...] = _dot(h2_, un0[...])
    zn1[...] = _dot(h2_, un1[...])
    onehot = (gid[...] == lax.broadcasted_iota(i32, (BN, NG), 1)).astype(f32)
    acc = _seg_dot(onehot, h2_)

    @pl.when(pl.program_id(0) == 0)
    def _():
        g0[...] = jnp.zeros_like(g0)

    g0[...] += acc


def _tc7a(wsum, h, gid, Wi, Wh, bi, bh, un0, un1):
    return pl.pallas_call(
        _tc7a_body,
        grid=(NP // BN,),
        in_specs=[_rows(BN, GP),
                  _rows(BN, GP), _rows(BN, 1),
                  _full((G, 3 * G)), _full((G, 3 * G)), _full((1, 3 * G)),
                  _full((1, 3 * G)), _full((G, 1)), _full((G, 1))],
        out_specs=[_rows(BN, G), _rows(BN, 1), _rows(BN, 1),
                   _full((NG, G))],
        out_shape=[jax.ShapeDtypeStruct((NP, G), f32),
                   jax.ShapeDtypeStruct((NP, 1), f32),
                   jax.ShapeDtypeStruct((NP, 1), f32),
                   jax.ShapeDtypeStruct((NG, G), f32)],
        name="tc7a_layer1_update",
    )(wsum, h, gid, Wi, Wh, bi, bh, un0, un1)


def _tc7b_body(h2, znt, gid, gprev, ug, bcl, Wpnt, bpnt, sg, gw):
    zn_g = _dot(jnp.maximum(gprev[...], 0.0), ug[...]) + bcl[...]
    onehot = (gid[...] == lax.broadcasted_iota(i32, (BN, NG), 1)).astype(f32)
    zg = _bcast_dot(onehot, zn_g)
    e = jnp.exp(_lrelu(zg + znt[...]))
    hp = _dot(h2[:, :G], Wpnt[...]) + bpnt[...]
    sg_acc = _seg_dot(onehot, e)
    gw_acc = _seg_dot(onehot, e * hp)

    @pl.when(pl.program_id(0) == 0)
    def _():
        sg[...] = jnp.zeros_like(sg)
        gw[...] = jnp.zeros_like(gw)

    sg[...] += sg_acc
    gw[...] += gw_acc


def _tc7b(h2, znt, gid, gprev, ug, bcl, Wpnt, bpnt):
    return pl.pallas_call(
        _tc7b_body,
        grid=(NP // BN,),
        in_specs=[_rows(BN, G), _rows(BN, 1), _rows(BN, 1), _full((NG, G)),
                  _full((G, 1)), _full((1, 1)), _full((G, G)),
                  _full((1, G))],
        out_specs=[_full((NG, 1)), _full((NG, G))],
        out_shape=[jax.ShapeDtypeStruct((NG, 1), f32),
                   jax.ShapeDtypeStruct((NG, G), f32)],
        name="tc7b_readout_attn",
    )(h2, znt, gid, gprev, ug, bcl, Wpnt, bpnt)


def _tc7c_body(sg, gw, gprev, Wi, Wh, bi, bh, gnext):
    gr = gw[...] / (sg[...] + 1e-9)
    gnext[...] = jnp.maximum(
        _gru_block(_elu(gr), gprev[...], Wi[...], Wh[...], bi[...], bh[...]),
        0.0)


def _tc7c(sg, gw, gprev, Wi, Wh, bi, bh):
    return pl.pallas_call(
        _tc7c_body,
        grid=(1,),
        in_specs=[_full((NG, 1)), _full((NG, G)), _full((NG, G)),
                  _full((G, 3 * G)), _full((G, 3 * G)), _full((1, 3 * G)),
                  _full((1, 3 * G))],
        out_specs=[_full((NG, G))],
        out_shape=[jax.ShapeDtypeStruct((NG, G), f32)],
        name="tc7c_graph_gru",
    )(sg, gw, gprev, Wi, Wh, bi, bh)[0]


def _head_body(sg, gw, gprev, Wi, Wh, bi, bh, Wp1, bp1, gamma, beta,
               Wp2, bp2, out):
    gr = gw[...] / (sg[...] + 1e-9)
    g2 = jnp.maximum(
        _gru_block(_elu(gr), gprev[...], Wi[...], Wh[...], bi[...], bh[...]),
        0.0)
    y = jnp.maximum(_dot(g2, Wp1[...]) + bp1[...], 0.0)
    mu = jnp.mean(y, axis=-1, keepdims=True)
    yc = y - mu
    var = jnp.mean(yc * yc, axis=-1, keepdims=True)
    y = yc * lax.rsqrt(var + 1e-5) * gamma[...] + beta[...]
    out[...] = _dot(y, Wp2[...]) + bp2[...]


def _head(sg, gw, gprev, Wi, Wh, bi, bh, Wp1, bp1, gamma, beta, Wp2, bp2):
    return pl.pallas_call(
        _head_body,
        grid=(1,),
        in_specs=[_full((NG, 1)), _full((NG, G)), _full((NG, G)),
                  _full((G, 3 * G)), _full((G, 3 * G)), _full((1, 3 * G)),
                  _full((1, 3 * G)), _full((G, PH)), _full((1, PH)),
                  _full((1, PH)), _full((1, PH)), _full((PH, 1)),
                  _full((1, 1))],
        out_specs=[_full((NG, 1))],
        out_shape=[jax.ShapeDtypeStruct((NG, 1), f32)],
        name="tc_head",
    )(sg, gw, gprev, Wi, Wh, bi, bh, Wp1, bp1, gamma, beta, Wp2, bp2)[0]


def _wid():
    return lax.axis_index("s") * NC + lax.axis_index("c")


def _chunk_loop(body):
    w = _wid()

    def step(j, _):
        c = w + j * NW

        @pl.when(c < NCHUNK)
        def _():
            body(c)
        return 0

    lax.fori_loop(0, STEPS, step, 0)


@functools.partial(
    pl.kernel,
    out_type=jax.ShapeDtypeStruct((E, GP), f32),
    mesh=_mesh,
    compiler_params=_sc_params_tiled,
    scratch_types=[pltpu.VMEM((CHUNK,), i32),
                   pltpu.VMEM((CHUNK,), i32),
                   pltpu.VMEM((CHUNK, GP), f32),
                   pltpu.VMEM((CHUNK, GP), f32),
                   pltpu.SemaphoreType.DMA,
                   pltpu.SemaphoreType.DMA],
    name="sc_gather_rows",
)
def _sc_gather_rows(table, idx, out, iv0, iv1, rv0, rv1, sm0, sm1):
    w = _wid()
    tsteps = (NCHUNK + NW - 1) // NW

    def issue(c, iv, rv, sm):
        pltpu.sync_copy(idx.at[pl.ds(c * CHUNK, CHUNK)], iv)
        pltpu.async_copy(table.at[iv], rv, sm)

    def process(j, iv, rv, sm, ivn, rvn, smn):
        c = w + j * NW

        @pl.when(c < NCHUNK)
        def _():
            cn = c + NW

            @pl.when(cn < NCHUNK)
            def _():
                issue(cn, ivn, rvn, smn)
            pltpu.make_async_copy(table.at[iv], rv, sm).wait()
            pltpu.sync_copy(rv, out.at[pl.ds(c * CHUNK, CHUNK)])

    issue(w, iv0, rv0, sm0)

    def step(jj, _):
        j = 2 * jj
        process(j, iv0, rv0, sm0, iv1, rv1, sm1)
        process(j + 1, iv1, rv1, sm1, iv0, rv0, sm0)
        return 0

    lax.fori_loop(0, (tsteps + 1) // 2, step, 0)


@functools.partial(
    pl.kernel,
    out_type=jax.ShapeDtypeStruct((E,), f32),
    mesh=_mesh,
    compiler_params=_sc_params,
    scratch_types=[pltpu.VMEM((NP,), f32),
                   pltpu.VMEM((CHUNK,), i32),
                   pltpu.VMEM((CHUNK,), f32)],
    name="sc_gather_scalar",
)
def _sc_gather_scalar(table, idx, out, table_v, idx_v, out_v):
    pltpu.sync_copy(table, table_v)

    def body(c):
        base = c * CHUNK
        pltpu.sync_copy(idx.at[pl.ds(base, CHUNK)], idx_v)
        for g in range(CHUNK // LANES):
            iv = idx_v[pl.ds(g * LANES, LANES)]
            out_v[pl.ds(g * LANES, LANES)] = plsc.load_gather(table_v, [iv])
        pltpu.sync_copy(out_v, out.at[pl.ds(base, CHUNK)])
    _chunk_loop(body)


@functools.partial(
    pl.kernel,
    out_type=jax.ShapeDtypeStruct((NP, GP), f32),
    mesh=_mesh,
    compiler_params=_sc_params_tiled,
    scratch_types=[pltpu.VMEM((CHUNK, GW), f32),
                   pltpu.VMEM((CHUNK, GW), f32),
                   pltpu.VMEM((1, CHUNK), i32),
                   pltpu.VMEM((1, CHUNK), i32),
                   pltpu.VMEM_SHARED((NP, GW), f32),
                   pltpu.SemaphoreType.DMA,
                   pltpu.SemaphoreType.DMA],
    name="sc_scatter_add_rows",
)
def _sc_scatter_add_rows(rows, idx, zblk, out, rv0, rv1, iv0, iv1, shared,
                         sm0, sm1):
    cid = lax.axis_index("c")
    sid = lax.axis_index("s")
    col = cid * GW
    tsteps = (NCHUNK + NS - 1) // NS

    pltpu.sync_copy(zblk, rv0)
    for k in range(RPS // CHUNK):
        pltpu.sync_copy(rv0,
                        shared.at[pl.ds(sid * RPS + k * CHUNK, CHUNK)])
    plsc.subcore_barrier()

    def issue(c, rv, iv, sm):
        base = c * CHUNK
        pltpu.sync_copy(idx.at[pl.ds(base, CHUNK)], iv.at[0])
        pltpu.async_copy(rows.at[pl.ds(base, CHUNK), pl.ds(col, GW)], rv, sm)

    def process(j, rv, iv, sm, rvn, ivn, smn):
        c = sid + j * NS

        @pl.when(c < NCHUNK)
        def _():
            cn = c + NS

            @pl.when(cn < NCHUNK)
            def _():
                issue(cn, rvn, ivn, smn)
            base = c * CHUNK
            pltpu.make_async_copy(
                rows.at[pl.ds(base, CHUNK), pl.ds(col, GW)], rv, sm).wait()
            pltpu.sync_copy(rv, shared.at[iv.at[0]], add=True)

    issue(sid, rv0, iv0, sm0)

    def step(jj, _):
        j = 2 * jj
        process(j, rv0, iv0, sm0, rv1, iv1, sm1)
        process(j + 1, rv1, iv1, sm1, rv0, iv0, sm0)
        return 0

    lax.fori_loop(0, (tsteps + 1) // 2, step, 0)

    plsc.subcore_barrier()
    pltpu.sync_copy(shared.at[pl.ds(sid * RPS, RPS)],
                    out.at[pl.ds(sid * RPS, RPS), pl.ds(col, GW)])


@functools.partial(
    pl.kernel,
    out_type=jax.ShapeDtypeStruct((NP, GP), f32),
    mesh=_mesh,
    compiler_params=_sc_params_tiled,
    scratch_types=[pltpu.VMEM((CHUNK, GW), f32),
                   pltpu.VMEM((CHUNK, GW), f32),
                   pltpu.VMEM((1, CHUNK), i32),
                   pltpu.VMEM((1, CHUNK), i32),
                   pltpu.VMEM((1, CHUNK), i32),
                   pltpu.VMEM((CHUNK,), f32),
                   pltpu.VMEM_SHARED((NP, GW), f32),
                   pltpu.SemaphoreType.DMA,
                   pltpu.SemaphoreType.DMA],
    name="sc_l1_aggregate",
)
def _sc_l1_aggregate(h_cat, elog2, src, dst, zblk, out, rv0, rv1, sv0, sv1,
                     di_v, e_v, shared, sm0, sm1):
    cid = lax.axis_index("c")
    sid = lax.axis_index("s")
    col = cid * GW
    row_off = cid * NP
    tsteps = (NCHUNK + NS - 1) // NS

    pltpu.sync_copy(zblk, rv0)
    for k in range(RPS // CHUNK):
        pltpu.sync_copy(rv0,
                        shared.at[pl.ds(sid * RPS + k * CHUNK, CHUNK)])
    plsc.subcore_barrier()

    def issue(c, rv, sv, sm):
        base = c * CHUNK
        pltpu.sync_copy(src.at[pl.ds(base, CHUNK)], sv.at[0])
        for g in range(CHUNK // LANES):
            sl = pl.ds(g * LANES, LANES)
            sv[0, sl] = sv[0, sl] + row_off
        pltpu.async_copy(h_cat.at[sv.at[0]], rv, sm)

    def process(j, rv, sv, sm, rvn, svn, smn):
        c = sid + j * NS

        @pl.when(c < NCHUNK)
        def _():
            cn = c + NS

            @pl.when(cn < NCHUNK)
            def _():
                issue(cn, rvn, svn, smn)
            base = c * CHUNK
            pltpu.sync_copy(dst.at[pl.ds(base, CHUNK)], di_v.at[0])
            pltpu.sync_copy(elog2.at[pl.ds(base, CHUNK)], e_v)
            pltpu.make_async_copy(h_cat.at[sv.at[0]], rv, sm).wait()

            def scale(r, _):
                bc = plsc.load_gather(e_v, [jnp.full((LANES,), r, i32)])
                for q in range(GW // LANES):
                    qs = pl.ds(q * LANES, LANES)
                    rv[r, qs] = rv[r, qs] * bc
                return 0

            lax.fori_loop(0, CHUNK, scale, 0)
            pltpu.sync_copy(rv, shared.at[di_v.at[0]], add=True)

    issue(sid, rv0, sv0, sm0)

    def step(jj, _):
        j = 2 * jj
        process(j, rv0, sv0, sm0, rv1, sv1, sm1)
        process(j + 1, rv1, sv1, sm1, rv0, sv0, sm0)
        return 0

    lax.fori_loop(0, (tsteps + 1) // 2, step, 0)

    plsc.subcore_barrier()
    pltpu.sync_copy(shared.at[pl.ds(sid * RPS, RPS)],
                    out.at[pl.ds(sid * RPS, RPS), pl.ds(col, GW)])


@functools.partial(
    pl.kernel,
    out_type=jax.ShapeDtypeStruct((E,), f32),
    mesh=_mesh,
    compiler_params=_sc_params,
    scratch_types=[pltpu.VMEM((NP,), f32),
                   pltpu.VMEM((NP,), f32),
                   pltpu.VMEM((CHUNK,), i32),
                   pltpu.VMEM((CHUNK,), i32),
                   pltpu.VMEM((CHUNK,), f32)],
    name="sc_layer1_logits",
)
def _sc_layer1_logits(sd2, ss2, dst, src, elog2, td_v, ts_v, di_v, si_v,
                      e_v):
    pltpu.sync_copy(sd2, td_v)
    pltpu.sync_copy(ss2, ts_v)

    def body(c):
        base = c * CHUNK
        pltpu.sync_copy(dst.at[pl.ds(base, CHUNK)], di_v)
        pltpu.sync_copy(src.at[pl.ds(base, CHUNK)], si_v)
        for g in range(CHUNK // LANES):
            sl = pl.ds(g * LANES, LANES)
            dv = di_v[sl]
            sv = si_v[sl]
            zd = plsc.load_gather(td_v, [dv])
            zs = plsc.load_gather(ts_v, [sv])
            z = zd + zs
            e = jnp.exp(jnp.where(z >= 0, z, 0.01 * z))
            e_v[sl] = e
        pltpu.sync_copy(e_v, elog2.at[pl.ds(base, CHUNK)])
    _chunk_loop(body)


def kernel(node_feats, edge_feats, edge_index, graph_ids, params):
    src = edge_index[0].astype(i32)
    dst = edge_index[1].astype(i32)

    nf_p = jnp.zeros((NP, F), f32).at[:N].set(node_feats)
    gid_p = jnp.full((NP,), NG, i32).at[:N].set(graph_ids.astype(i32))
    gid_col = gid_p.reshape(NP, 1)

    p = params
    Wpn, bpn = p['pn'][0], p['pn'][1].reshape(1, G)
    Wpe1, bpe1 = p['pe1'][0], p['pe1'][1].reshape(1, G)
    A, Bm = Wpe1[:F], Wpe1[F:]
    Wpe2, bpe2 = p['pe2'][0], p['pe2'][1].reshape(1, 1)
    wtop, wbot = Wpe2[:G], Wpe2[G:]
    gpad = ((0, 0), (0, GP - G))
    A_pad = jnp.pad(A, gpad)
    Bm_pad = jnp.pad(Bm, gpad)
    bpe1_pad = jnp.pad(bpe1, gpad)
    wbot_pad = jnp.pad(wbot, ((0, GP - G), (0, 0)))
    Wet, bet = p['et'][0], p['et'][1].reshape(1, G)
    Wet_pad = jnp.pad(Wet, ((0, GP - G), (0, GP - G)))
    bet_pad = jnp.pad(bet, ((0, 0), (0, GP - G)))
    g1Wi, g1Wh, g1bi, g1bh = (p['gru1'][0], p['gru1'][1],
                              p['gru1'][2].reshape(1, 3 * G),
                              p['gru1'][3].reshape(1, 3 * G))
    Wl1pe, bl1pe = p['l1_pe'][0], p['l1_pe'][1].reshape(1, 1)
    vd, vs = Wl1pe[:G], Wl1pe[G:]
    Wl1pn, bl1pn = p['l1_pn'][0], p['l1_pn'][1].reshape(1, G)
    g2Wi, g2Wh, g2bi, g2bh = (p['l1_gru'][0], p['l1_gru'][1],
                              p['l1_gru'][2].reshape(1, 3 * G),
                              p['l1_gru'][3].reshape(1, 3 * G))

    hv, np1, sn = _tc1(nf_p, Wpn, bpn, A_pad, wtop, bpe2)
    gnp1 = _sc_gather_rows(np1, src)
    snd = _sc_gather_scalar(sn.reshape(NP), dst)
    w = _tc3(gnp1, edge_feats, snd.reshape(E, 1), Bm_pad, bpe1_pad,
             wbot_pad, Wet_pad, bet_pad)
    zblk = jnp.zeros((CHUNK, GW), f32)
    wsum = _sc_scatter_add_rows(w, dst, zblk)
    h, hp2, sd2, ss2 = _tc5(wsum, hv, g1Wi, g1Wh, g1bi, g1bh, vd, vs,
                            bl1pe, Wl1pn, bl1pn)

    elog2 = _sc_layer1_logits(sd2.reshape(NP), ss2.reshape(NP), dst, src)
    h_cat = jnp.concatenate([hp2[:, :GW], hp2[:, GW:]], axis=0)
    w2sum = _sc_l1_aggregate(h_cat, elog2, src, dst, zblk)

    ro0cl, bro0cl = p['ro0_cl'][0], p['ro0_cl'][1].reshape(1, 1)
    ug0, un0 = ro0cl[:G], ro0cl[G:]
    ro1cl, bro1cl = p['ro1_cl'][0], p['ro1_cl'][1].reshape(1, 1)
    ug1, un1 = ro1cl[:G], ro1cl[G:]
    W0pn, b0pn = p['ro0_pn'][0], p['ro0_pn'][1].reshape(1, G)
    W1pn, b1pn = p['ro1_pn'][0], p['ro1_pn'][1].reshape(1, G)
    r0Wi, r0Wh, r0bi, r0bh = (p['ro0_gru'][0], p['ro0_gru'][1],
                              p['ro0_gru'][2].reshape(1, 3 * G),
                              p['ro0_gru'][3].reshape(1, 3 * G))
    r1Wi, r1Wh, r1bi, r1bh = (p['ro1_gru'][0], p['ro1_gru'][1],
                              p['ro1_gru'][2].reshape(1, 3 * G),
                              p['ro1_gru'][3].reshape(1, 3 * G))
    Wp1, bp1 = p['p1'][0], p['p1'][1].reshape(1, PH)
    gamma, beta = p['ln'][0].reshape(1, PH), p['ln'][1].reshape(1, PH)
    Wp2, bp2 = p['p2'][0], p['p2'][1].reshape(1, 1)

    h2, zn0, zn1, g0 = _tc7a(w2sum, h, gid_col,
                             g2Wi, g2Wh, g2bi, g2bh, un0, un1)
    sg0, gw0 = _tc7b(h2, zn0, gid_col, g0, ug0, bro0cl, W0pn, b0pn)
    g1 = _tc7c(sg0, gw0, g0, r0Wi, r0Wh, r0bi, r0bh)
    sg1, gw1 = _tc7b(h2, zn1, gid_col, g1, ug1, bro1cl, W1pn, b1pn)
    out = _head(sg1, gw1, g1, r1Wi, r1Wh, r1bi, r1bh,
                Wp1, bp1, gamma, beta, Wp2, bp2)
    return out
